# baseline (speedup 1.0000x reference)
import functools

import jax
import jax.numpy as jnp
from jax import lax
from jax.experimental import pallas as pl
from jax.experimental.pallas import tpu as pltpu
from jax.experimental.pallas import tpu_sc as plsc

_IDX_MID = (2, 0, 0, 1, 1, 3, 3, 5, 4, 5, 4, 7, 6, 7, 6, 9, 8, 11, 11, 9,
            10, 10, 8, 12, 12, 13, 13, 14, 14, 15, 15, 16, 17, 17, 18, 18,
            19, 19, 20, 20, 21, 21, 22, 22, 23, 23, 24, 24, 25, 25, 26, 26,
            27, 27, 28, 28, 29, 30, 30, 31, 32, 32, 31, 33, 33, 34, 35, 35,
            34, 36, 36, 37, 37, 38, 29, 38, 39, 39, 40, 40, 16, 41, 41, 42,
            42, 43, 43, 44, 44, 45, 45, 46, 47, 47, 48, 48, 49, 49, 50, 50,
            51, 51, 52, 52, 53, 53, 54, 54, 55, 55, 56, 56, 57, 58, 58, 59,
            59, 60, 60, 61, 61, 62, 62, 63, 63, 64, 64, 65, 65, 66, 66, 67,
            67)

_IDX_LOW = (0, 0, 1, 1, 2, 2, 3, 3, 4, 4, 5, 5, 6, 7, 8, 9, 10, 9, 8, 7,
            6, 11, 12, 13, 12, 11, 13, 14, 15, 14, 15, 16, 17, 18, 16, 17,
            18, 19, 10, 19, 20, 20, 21, 21, 22, 22, 23, 24, 25, 26, 27, 28,
            29, 30, 31, 32, 33, 23, 24, 25, 26, 27, 28, 29, 30, 31, 32, 33)


@functools.lru_cache(maxsize=None)
def _make_unpool(batch, j_in, d, idx):
    j_out = len(idx)
    info = plsc.get_sparse_core_info()
    nw = info.num_cores * info.num_subcores
    bb = 8
    assert batch % (nw * bb) == 0
    passes = batch // (nw * bb)

    @functools.partial(
        pl.kernel,
        out_type=jax.ShapeDtypeStruct((batch, j_out, d), jnp.float32),
        mesh=plsc.VectorSubcoreMesh(core_axis_name="c", subcore_axis_name="s"),
        scratch_types=[
            pltpu.VMEM((bb, j_in, d), jnp.float32),
            pltpu.SemaphoreType.DMA,
        ],
    )
    def unpool(x_hbm, out_hbm, slab, sem):
        wid = lax.axis_index("s") * info.num_cores + lax.axis_index("c")

        def body(p, carry):
            b0 = (p * nw + wid) * bb
            pltpu.sync_copy(x_hbm.at[pl.ds(b0, bb)], slab)
            copies = [
                pltpu.async_copy(
                    slab.at[:, pl.ds(idx[j], 1), :],
                    out_hbm.at[pl.ds(b0, bb), pl.ds(j, 1), :],
                    sem,
                )
                for j in range(j_out)
            ]
            for c in copies:
                c.wait()
            return carry

        lax.fori_loop(0, passes, body, 0)

    return unpool


def kernel(x):
    batch, j_in, d = x.shape
    idx = _IDX_MID if j_in == 68 else _IDX_LOW
    return _make_unpool(batch, j_in, d, idx)(x)
